# baseline (speedup 1.0000x reference)
import jax
import jax.numpy as jnp
from jax import lax
from jax.experimental import pallas as pl
from jax.experimental.pallas import tpu as pltpu
from jax.experimental.pallas import tpu_sc as plsc

VOCAB = 128
D = 128
B = 16384
NC = 2
NS = 16
NW = NC * NS
BPW = B // NW
CHUNK = 256
NCHUNK = BPW // CHUNK


def _body(pos_hbm, row_hbm, col_hbm, out_hbm,
          pos_v, rtab_v, ctab_v, ridx_v, cidx_v, out_v, sem_in, sem_out):
    wid = lax.axis_index("s") * NC + lax.axis_index("c")
    base = wid * BPW

    cps = [pltpu.async_copy(row_hbm, rtab_v, sem_in),
           pltpu.async_copy(col_hbm, ctab_v, sem_in)]
    for a in range(4):
        cps.append(
            pltpu.async_copy(pos_hbm.at[a, pl.ds(base, BPW)], pos_v.at[a],
                             sem_in))

    for cp in cps:
        cp.wait()

    @plsc.parallel_loop(0, BPW // 16)
    def idx_body(j):
        s = pl.ds(j * 16, 16)
        qr0 = jnp.minimum((pos_v[0, s] * VOCAB).astype(jnp.int32), VOCAB - 1)
        qc0 = jnp.minimum((pos_v[1, s] * VOCAB).astype(jnp.int32), VOCAB - 1)
        qr1 = jnp.minimum((pos_v[2, s] * VOCAB).astype(jnp.int32), VOCAB - 1)
        qc1 = jnp.minimum((pos_v[3, s] * VOCAB).astype(jnp.int32), VOCAB - 1)
        ridx_v[s] = jnp.right_shift(qr0 + qr1, 1)
        cidx_v[s] = jnp.right_shift(qc0 + qc1, 1)

    out_cps = [None, None]
    for c in range(NCHUNK):
        buf = c % 2
        if out_cps[buf] is not None:
            out_cps[buf].wait()

        @plsc.parallel_loop(0, CHUNK // 16, unroll=2)
        def row_body(g):
            rvec = ridx_v[pl.ds(c * CHUNK + g * 16, 16)]
            cvec = cidx_v[pl.ds(c * CHUNK + g * 16, 16)]
            for e in range(16):
                ri = rvec[e]
                ci = cvec[e]
                for k in range(D // 16):
                    s = pl.ds(k * 16, 16)
                    out_v[buf, g * 16 + e, s] = rtab_v[ri, s] + ctab_v[ci, s]

        out_cps[buf] = pltpu.async_copy(
            out_v.at[buf], out_hbm.at[pl.ds(base + c * CHUNK, CHUNK)],
            sem_out)

    for cp in out_cps:
        if cp is not None:
            cp.wait()


_mesh = plsc.VectorSubcoreMesh(core_axis_name="c", subcore_axis_name="s")

_kern = pl.kernel(
    _body,
    out_type=jax.ShapeDtypeStruct((B, D), jnp.float32),
    mesh=_mesh,
    scratch_types=[
        pltpu.VMEM((4, BPW), jnp.float32),
        pltpu.VMEM((VOCAB, D), jnp.float32),
        pltpu.VMEM((VOCAB, D), jnp.float32),
        pltpu.VMEM((BPW,), jnp.int32),
        pltpu.VMEM((BPW,), jnp.int32),
        pltpu.VMEM((2, CHUNK, D), jnp.float32),
        pltpu.SemaphoreType.DMA,
        pltpu.SemaphoreType.DMA,
    ],
)


def kernel(patch_positions, row_embedding, column_embedding):
    pos_planes = patch_positions.reshape(B, 4).T
    return _kern(pos_planes, row_embedding, column_embedding)


# speedup vs baseline: 1.4612x; 1.4612x over previous
import jax
import jax.numpy as jnp
from jax import lax
from jax.experimental import pallas as pl
from jax.experimental.pallas import tpu as pltpu
from jax.experimental.pallas import tpu_sc as plsc

VOCAB = 128
D = 128
B = 16384
NC = 2
NS = 16
NW = NC * NS
BPW = B // NW
CHUNK = 256
NCHUNK = BPW // CHUNK


def _body(pos_hbm, row_hbm, col_hbm, out_hbm,
          pos_v, rtab_v, ctab_v, ridx_v, cidx_v, out_v, sem_in, sem_out):
    wid = lax.axis_index("s") * NC + lax.axis_index("c")
    base = wid * BPW

    cps = [pltpu.async_copy(row_hbm, rtab_v, sem_in),
           pltpu.async_copy(col_hbm, ctab_v, sem_in)]
    for a in range(4):
        cps.append(
            pltpu.async_copy(pos_hbm.at[a, pl.ds(base, BPW)], pos_v.at[a],
                             sem_in))

    for cp in cps:
        cp.wait()

    @plsc.parallel_loop(0, BPW // 16)
    def idx_body(j):
        s = pl.ds(j * 16, 16)
        qr0 = jnp.minimum((pos_v[0, s] * VOCAB).astype(jnp.int32), VOCAB - 1)
        qc0 = jnp.minimum((pos_v[1, s] * VOCAB).astype(jnp.int32), VOCAB - 1)
        qr1 = jnp.minimum((pos_v[2, s] * VOCAB).astype(jnp.int32), VOCAB - 1)
        qc1 = jnp.minimum((pos_v[3, s] * VOCAB).astype(jnp.int32), VOCAB - 1)
        ridx_v[s] = jnp.right_shift(qr0 + qr1, 1)
        cidx_v[s] = jnp.right_shift(qc0 + qc1, 1)

    out_cps = [None, None]
    for c in range(NCHUNK):
        buf = c % 2
        if out_cps[buf] is not None:
            out_cps[buf].wait()

        @plsc.parallel_loop(0, CHUNK // 16)
        def row_body(g):
            rvec = ridx_v[pl.ds(c * CHUNK + g * 16, 16)]
            cvec = cidx_v[pl.ds(c * CHUNK + g * 16, 16)]
            for e in range(16):
                ri = rvec[e]
                ci = cvec[e]
                rparts = [rtab_v[ri, pl.ds(k * 16, 16)] for k in range(D // 16)]
                cparts = [ctab_v[ci, pl.ds(k * 16, 16)] for k in range(D // 16)]
                for k in range(D // 16):
                    out_v[buf, g * 16 + e, pl.ds(k * 16, 16)] = (
                        rparts[k] + cparts[k])

        out_cps[buf] = pltpu.async_copy(
            out_v.at[buf], out_hbm.at[pl.ds(base + c * CHUNK, CHUNK)],
            sem_out)

    for cp in out_cps:
        if cp is not None:
            cp.wait()


_mesh = plsc.VectorSubcoreMesh(core_axis_name="c", subcore_axis_name="s")

_kern = pl.kernel(
    _body,
    out_type=jax.ShapeDtypeStruct((B, D), jnp.float32),
    mesh=_mesh,
    scratch_types=[
        pltpu.VMEM((4, BPW), jnp.float32),
        pltpu.VMEM((VOCAB, D), jnp.float32),
        pltpu.VMEM((VOCAB, D), jnp.float32),
        pltpu.VMEM((BPW,), jnp.int32),
        pltpu.VMEM((BPW,), jnp.int32),
        pltpu.VMEM((2, CHUNK, D), jnp.float32),
        pltpu.SemaphoreType.DMA,
        pltpu.SemaphoreType.DMA,
    ],
)


def kernel(patch_positions, row_embedding, column_embedding):
    pos_planes = patch_positions.reshape(B, 4).T
    return _kern(pos_planes, row_embedding, column_embedding)
